# Initial kernel scaffold; baseline (speedup 1.0000x reference)
import jax
import jax.numpy as jnp
from jax.experimental import pallas as pl


def kernel(x, crop_x, crop_y, b):
    raise NotImplementedError("write your pallas kernel here")


# speedup vs baseline: 11.2466x; 11.2466x over previous
import jax
import jax.numpy as jnp
from jax import lax
from jax.experimental import pallas as pl
from jax.experimental.pallas import tpu as pltpu
from jax.experimental.pallas import tpu_sc as plsc

_NBUF = 4
_ROW = 8192
_ROWS_PER_WORKER = 128
_TBL = 32768


def _sc_body(x_hbm, tbl_hbm, starts_hbm, out_hbm,
             tbl_v, starts_v,
             in0, in1, in2, in3, ou0, ou1, ou2, ou3, in_sems, out_sems):
    in_bufs = [in0, in1, in2, in3]
    out_bufs = [ou0, ou1, ou2, ou3]
    num_cores = 2
    wid = lax.axis_index("s") * num_cores + lax.axis_index("c")
    r0 = wid * _ROWS_PER_WORKER
    n = r0 // 512
    j0 = r0 % 512

    pltpu.sync_copy(starts_hbm, starts_v)
    pltpu.sync_copy(tbl_hbm, tbl_v)

    lane = lax.iota(jnp.int32, 16)
    nvec = (lane * 0 + n)[:, None]
    dn = lax.GatherDimensionNumbers(
        offset_dims=(), collapsed_slice_dims=(0,), start_index_map=(0,))
    base_vec = lax.gather(starts_v[...], nvec, dn, slice_sizes=(1,),
                          mode=lax.GatherScatterMode.PROMISE_IN_BOUNDS)
    base_vec = base_vec - j0 * 16 + lane

    def start_in(b, t):
        return pltpu.async_copy(x_hbm.at[r0 + t], in_bufs[b], in_sems.at[b])

    def start_out(b, t):
        return pltpu.async_copy(out_bufs[b], out_hbm.at[r0 + t],
                                out_sems.at[b])

    def wait_in(b, t):
        pltpu.make_async_copy(x_hbm.at[r0 + t], in_bufs[b],
                              in_sems.at[b]).wait()

    def wait_out(b, t):
        pltpu.make_async_copy(out_bufs[b], out_hbm.at[r0 + t],
                              out_sems.at[b]).wait()

    for b in range(_NBUF):
        start_in(b, b)

    @pl.loop(0, _ROWS_PER_WORKER, step=_NBUF)
    def _row_group(t0):
        for b in range(_NBUF):
            t = t0 + b
            wait_in(b, t)

            @pl.when(t >= _NBUF)
            def _():
                wait_out(b, t - _NBUF)

            src = in_bufs[b]
            dst = out_bufs[b]
            row_vec = base_vec - t * 16

            @plsc.parallel_loop(0, _ROW // 16, unroll=8)
            def _chunk(k):
                i = k * 16
                tval = plsc.load_gather(tbl_v, [row_vec + i])
                dst[pl.ds(i, 16)] = src[pl.ds(i, 16)] + tval

            start_out(b, t)

            @pl.when(t + _NBUF < _ROWS_PER_WORKER)
            def _():
                start_in(b, t + _NBUF)

    for b in range(_NBUF):
        wait_out(b, _ROWS_PER_WORKER - _NBUF + b)


def kernel(x, crop_x, crop_y, b):
    bsz, csx, csy, nb = x.shape
    bias_size = b.shape[0]

    table = jnp.concatenate(
        [b[1:][::-1], b, jnp.zeros((1, nb), b.dtype)], axis=0)
    tbl_flat = table.reshape(-1)

    start_diag = crop_x[:, 0].astype(jnp.int32) - crop_y[:, 0].astype(jnp.int32)
    starts = (start_diag + bias_size - 1) * nb
    starts = jnp.concatenate(
        [starts, jnp.zeros((16 - bsz,), jnp.int32)])

    xf = x.reshape(bsz * csx, csy * nb)

    mesh = plsc.VectorSubcoreMesh(core_axis_name="c", subcore_axis_name="s")
    out = pl.kernel(
        _sc_body,
        out_type=jax.ShapeDtypeStruct(xf.shape, x.dtype),
        mesh=mesh,
        compiler_params=pltpu.CompilerParams(needs_layout_passes=False),
        scratch_types=[
            pltpu.VMEM((_TBL,), jnp.float32),
            pltpu.VMEM((16,), jnp.int32),
            pltpu.VMEM((_ROW,), jnp.float32),
            pltpu.VMEM((_ROW,), jnp.float32),
            pltpu.VMEM((_ROW,), jnp.float32),
            pltpu.VMEM((_ROW,), jnp.float32),
            pltpu.VMEM((_ROW,), jnp.float32),
            pltpu.VMEM((_ROW,), jnp.float32),
            pltpu.VMEM((_ROW,), jnp.float32),
            pltpu.VMEM((_ROW,), jnp.float32),
            pltpu.SemaphoreType.DMA((_NBUF,)),
            pltpu.SemaphoreType.DMA((_NBUF,)),
        ],
    )(xf, tbl_flat, starts)
    return out.reshape(bsz, csx, csy, nb)
